# baseline (speedup 1.0000x reference)
import functools

import jax
import jax.numpy as jnp
from jax import lax
from jax.experimental import pallas as pl
from jax.experimental.pallas import tpu as pltpu
from jax.experimental.pallas import tpu_sc as plsc

N = 10000
E = 320000
D_IN = 128
D_EDGE = 16
H = 8
C = 16
HC = H * C
N_REL = 64

ACC_W = 144
K = 80
N_CHUNKS = E // 32 // K
RPT = N // 16
ROW_BLK = 1000


def _pre_body(x_ref, wl_ref, bl_ref, wr_ref, br_ref, rel_ref, we_ref,
              xl_ref, xr_ref, relp_ref):
    xb = x_ref[...]
    xl_ref[...] = jnp.dot(xb, wl_ref[...],
                          preferred_element_type=jnp.float32) + bl_ref[...]
    xr_ref[...] = jnp.dot(xb, wr_ref[...],
                          preferred_element_type=jnp.float32) + br_ref[...]
    relp_ref[...] = jnp.dot(rel_ref[...], we_ref[...],
                            preferred_element_type=jnp.float32)


def _pre(x, Wl, bl2, Wr, br2, relations, We):
    grid = (N // ROW_BLK,)
    return pl.pallas_call(
        _pre_body,
        grid=grid,
        in_specs=[
            pl.BlockSpec((ROW_BLK, D_IN), lambda i: (i, 0)),
            pl.BlockSpec((D_IN, HC), lambda i: (0, 0)),
            pl.BlockSpec((1, HC), lambda i: (0, 0)),
            pl.BlockSpec((D_IN, HC), lambda i: (0, 0)),
            pl.BlockSpec((1, HC), lambda i: (0, 0)),
            pl.BlockSpec((N_REL, D_EDGE), lambda i: (0, 0)),
            pl.BlockSpec((D_EDGE, HC), lambda i: (0, 0)),
        ],
        out_specs=[
            pl.BlockSpec((ROW_BLK, HC), lambda i: (i, 0)),
            pl.BlockSpec((ROW_BLK, HC), lambda i: (i, 0)),
            pl.BlockSpec((N_REL, HC), lambda i: (0, 0)),
        ],
        out_shape=[
            jax.ShapeDtypeStruct((N, HC), jnp.float32),
            jax.ShapeDtypeStruct((N, HC), jnp.float32),
            jax.ShapeDtypeStruct((N_REL, HC), jnp.float32),
        ],
    )(x, Wl, bl2, Wr, br2, relations, We)


def _zero_acc(contrib, acc_sh, sid, width, rows):
    zero16 = jnp.zeros((16,), jnp.float32)

    def _zrow(r, carry):
        for j in range(width // 16):
            contrib[r, pl.ds(16 * j, 16)] = zero16
        return carry
    lax.fori_loop(0, rows, _zrow, 0)
    base = sid * RPT
    for j in range(RPT // rows):
        pltpu.sync_copy(contrib, acc_sh.at[pl.ds(base + j * rows, rows)])
    rem = RPT % rows
    if rem:
        pltpu.sync_copy(contrib.at[pl.ds(0, rem)],
                        acc_sh.at[pl.ds(base + (RPT // rows) * rows, rem)])


def _edge_body(xl_hbm, xr_hbm, relp_hbm, att_hbm,
               src_hbm, dst_hbm, rel_hbm, acc_out,
               src_v, dst_v, rel_v, xlg, xrbuf, contrib,
               relp_v, att_v, acc_sh, sema, semb, semi, semj, semk, sems):
    cid = lax.axis_index("c")
    sid = lax.axis_index("s")
    wid = cid * 16 + sid
    ebase = wid * (E // 32)
    zero16 = jnp.zeros((16,), jnp.float32)

    pltpu.sync_copy(att_hbm, att_v)
    pltpu.sync_copy(relp_hbm, relp_v)

    _zero_acc(contrib, acc_sh, sid, ACC_W, K)
    zero16i = jnp.zeros((16,), jnp.int32)
    for j in range(2):
        for b in range(K // 16):
            dst_v[j, pl.ds(16 * b, 16)] = zero16i
    plsc.subcore_barrier()

    pltpu.async_copy(src_hbm.at[pl.ds(ebase, K)], src_v.at[0], semi)
    pltpu.async_copy(dst_hbm.at[pl.ds(ebase, K)], dst_v.at[0], semj)
    pltpu.async_copy(rel_hbm.at[pl.ds(ebase, K)], rel_v.at[0], semk)
    pltpu.async_copy(contrib, acc_sh.at[dst_v.at[1]], sems, add=True)

    def _chunk(ci, carry):
        par = lax.rem(ci, 2)
        nxt = lax.rem(ci + 1, 2)
        pltpu.make_async_copy(src_hbm.at[pl.ds(ebase, K)],
                              src_v.at[par], semi).wait()
        pltpu.make_async_copy(dst_hbm.at[pl.ds(ebase, K)],
                              dst_v.at[par], semj).wait()
        pltpu.make_async_copy(rel_hbm.at[pl.ds(ebase, K)],
                              rel_v.at[par], semk).wait()
        cp_l = pltpu.async_copy(xl_hbm.at[src_v.at[par]], xlg, semb)
        cp_r = pltpu.async_copy(xr_hbm.at[dst_v.at[par]], xrbuf, sema)
        pltpu.make_async_copy(contrib, acc_sh.at[dst_v.at[nxt]], sems).wait()
        off = ebase + (ci + 1) * K
        safe = jnp.where(ci + 1 < N_CHUNKS, off, ebase)
        pltpu.async_copy(src_hbm.at[pl.ds(safe, K)], src_v.at[nxt], semi)
        pltpu.async_copy(dst_hbm.at[pl.ds(safe, K)], dst_v.at[nxt], semj)
        pltpu.async_copy(rel_hbm.at[pl.ds(safe, K)], rel_v.at[nxt], semk)
        cp_r.wait()
        cp_l.wait()

        lanes = lax.iota(jnp.int32, 16)

        ones16 = jnp.ones((16,), jnp.float32)

        def _group(g, gcarry):
            relg = rel_v[par, pl.ds(g * 16, 16)]
            for k in range(16):
                ek = g * 16 + k
                relk = relg[k]
                exvec = ones16
                for h in range(H):
                    ds = pl.ds(16 * h, 16)
                    x_h = xlg[ek, ds]
                    e = x_h + xrbuf[ek, ds] + relp_v[relk, ds]
                    e = jnp.where(e < 0.0, e * 0.2, e)
                    exb = jnp.exp(jnp.full((16,), jnp.sum(e * att_v[h, :]),
                                           jnp.float32))
                    contrib[ek, ds] = exb * x_h
                    exvec = jnp.where(lanes == h, exb, exvec)
                contrib[ek, pl.ds(128, 16)] = exvec
            return gcarry

        lax.fori_loop(0, K // 16, _group, 0)
        pltpu.async_copy(contrib, acc_sh.at[dst_v.at[par]], sems, add=True)
        return carry

    lax.fori_loop(0, N_CHUNKS, _chunk, 0)
    pltpu.make_async_copy(contrib, acc_sh.at[dst_v.at[0]], sems).wait()
    pltpu.make_async_copy(src_hbm.at[pl.ds(ebase, K)], src_v.at[0], semi).wait()
    pltpu.make_async_copy(dst_hbm.at[pl.ds(ebase, K)], dst_v.at[0], semj).wait()
    pltpu.make_async_copy(rel_hbm.at[pl.ds(ebase, K)], rel_v.at[0], semk).wait()
    plsc.subcore_barrier()

    pltpu.sync_copy(acc_sh.at[pl.ds(sid * RPT, RPT)],
                    acc_out.at[cid, pl.ds(sid * RPT, RPT)])


def _edge_pass(xl, xr, relp, att, src, dst, rel):
    mesh = plsc.VectorSubcoreMesh(core_axis_name="c", subcore_axis_name="s")
    f = functools.partial(
        pl.kernel,
        out_type=jax.ShapeDtypeStruct((2, N, ACC_W), jnp.float32),
        mesh=mesh,
        scratch_types=[
            pltpu.VMEM((2, K), jnp.int32),
            pltpu.VMEM((2, K), jnp.int32),
            pltpu.VMEM((2, K), jnp.int32),
            pltpu.VMEM((K, HC), jnp.float32),
            pltpu.VMEM((K, HC), jnp.float32),
            pltpu.VMEM((K, ACC_W), jnp.float32),
            pltpu.VMEM((N_REL, HC), jnp.float32),
            pltpu.VMEM((H, 16), jnp.float32),
            pltpu.VMEM_SHARED((N, ACC_W), jnp.float32),
            pltpu.SemaphoreType.DMA,
            pltpu.SemaphoreType.DMA,
            pltpu.SemaphoreType.DMA,
            pltpu.SemaphoreType.DMA,
            pltpu.SemaphoreType.DMA,
            pltpu.SemaphoreType.DMA,
        ],
        compiler_params=pltpu.CompilerParams(needs_layout_passes=False,
                                             use_tc_tiling_on_sc=False),
    )(_edge_body)
    return f(xl, xr, relp, att, src, dst, rel)


K2 = 400
SCW = 80


def _attr_body(relraw_hbm, dst_hbm, rel_hbm, acc2_out,
               dst_v2, rel_v, contrib, relraw_v, acc2_sh, semi, semr, sems):
    cid = lax.axis_index("c")
    sid = lax.axis_index("s")
    wid = cid * 16 + sid
    ebase = wid * (E // 32)
    nsc = K2 // SCW
    nchunks = E // 32 // K2
    zero16 = jnp.zeros((16,), jnp.float32)
    zero16i = jnp.zeros((16,), jnp.int32)

    pltpu.sync_copy(relraw_hbm, relraw_v)
    def _zrow(r, carry):
        contrib[0, r, :] = zero16
        contrib[1, r, :] = zero16
        return carry
    lax.fori_loop(0, K2, _zrow, 0)
    for j in range(nsc):
        for b in range(SCW // 16):
            dst_v2[1, j, pl.ds(16 * b, 16)] = zero16i
    base = sid * RPT
    pltpu.sync_copy(contrib.at[0], acc2_sh.at[pl.ds(base, K2)])
    pltpu.sync_copy(contrib.at[0, pl.ds(0, RPT - K2)],
                    acc2_sh.at[pl.ds(base + K2, RPT - K2)])
    plsc.subcore_barrier()

    for j in range(nsc):
        pltpu.async_copy(dst_hbm.at[pl.ds(ebase + j * SCW, SCW)],
                         dst_v2.at[0, j], semi)
        pltpu.async_copy(contrib.at[1, pl.ds(j * SCW, SCW)],
                         acc2_sh.at[dst_v2.at[1, j]], sems, add=True)
    pltpu.async_copy(rel_hbm.at[pl.ds(ebase, K2)], rel_v.at[0], semr)

    def _chunk(ci, carry):
        par = lax.rem(ci, 2)
        nxt = lax.rem(ci + 1, 2)
        pltpu.make_async_copy(rel_hbm.at[pl.ds(ebase, K2)],
                              rel_v.at[par], semr).wait()
        for j in range(nsc):
            pltpu.make_async_copy(dst_hbm.at[pl.ds(ebase, SCW)],
                                  dst_v2.at[par, j], semi).wait()

        def _group(g, gcarry):
            relg = rel_v[par, pl.ds(g * 16, 16)]
            for k in range(16):
                contrib[par, g * 16 + k, :] = relraw_v[relg[k], :]
            return gcarry

        lax.fori_loop(0, K2 // 16, _group, 0)
        for j in range(nsc):
            pltpu.make_async_copy(contrib.at[nxt, pl.ds(j * SCW, SCW)],
                                  acc2_sh.at[dst_v2.at[nxt, j]], sems).wait()
        off = ebase + (ci + 1) * K2
        safe = jnp.where(ci + 1 < nchunks, off, ebase)
        for j in range(nsc):
            pltpu.async_copy(contrib.at[par, pl.ds(j * SCW, SCW)],
                             acc2_sh.at[dst_v2.at[par, j]], sems, add=True)
            pltpu.async_copy(dst_hbm.at[pl.ds(safe + j * SCW, SCW)],
                             dst_v2.at[nxt, j], semi)
        pltpu.async_copy(rel_hbm.at[pl.ds(safe, K2)], rel_v.at[nxt], semr)
        return carry

    lax.fori_loop(0, nchunks, _chunk, 0)
    for j in range(nsc):
        pltpu.make_async_copy(contrib.at[0, pl.ds(j * SCW, SCW)],
                              acc2_sh.at[dst_v2.at[0, j]], sems).wait()
        pltpu.make_async_copy(dst_hbm.at[pl.ds(ebase, SCW)],
                              dst_v2.at[0, j], semi).wait()
    pltpu.make_async_copy(rel_hbm.at[pl.ds(ebase, K2)],
                          rel_v.at[0], semr).wait()
    plsc.subcore_barrier()
    pltpu.sync_copy(acc2_sh.at[pl.ds(sid * RPT, RPT)],
                    acc2_out.at[cid, pl.ds(sid * RPT, RPT)])


def _attr_pass(relations, dst, rel):
    mesh = plsc.VectorSubcoreMesh(core_axis_name="c", subcore_axis_name="s")
    f = functools.partial(
        pl.kernel,
        out_type=jax.ShapeDtypeStruct((2, N, D_EDGE), jnp.float32),
        mesh=mesh,
        scratch_types=[
            pltpu.VMEM((2, K2 // SCW, SCW), jnp.int32),
            pltpu.VMEM((2, K2), jnp.int32),
            pltpu.VMEM((2, K2, D_EDGE), jnp.float32),
            pltpu.VMEM((N_REL, D_EDGE), jnp.float32),
            pltpu.VMEM_SHARED((N, D_EDGE), jnp.float32),
            pltpu.SemaphoreType.DMA,
            pltpu.SemaphoreType.DMA,
            pltpu.SemaphoreType.DMA,
        ],
        compiler_params=pltpu.CompilerParams(needs_layout_passes=False,
                                             use_tc_tiling_on_sc=False),
    )(_attr_body)
    return f(relations, dst, rel)


def _combine_body(acc_ref, acc2_ref, xl_ref, xr_ref, we_ref, attf_ref,
                  bout_ref, out_ref):
    a0 = acc_ref[0]
    a1 = acc_ref[1]
    msg = a0[:, 0:128] + a1[:, 0:128]
    exs = a0[:, 128:136] + a1[:, 128:136]
    attr = acc2_ref[0] + acc2_ref[1]
    deg = a0[:, 136:137] + a1[:, 136:137]

    loop_attr = attr / jnp.maximum(deg, 1.0)
    loop_proj = jnp.dot(loop_attr, we_ref[...],
                        preferred_element_type=jnp.float32)
    e = xl_ref[...] + xr_ref[...] + loop_proj
    e = jnp.where(e < 0.0, e * 0.2, e)
    ea = e * attf_ref[...]

    hrow = lax.broadcasted_iota(jnp.int32, (HC, H), 0) // C
    hcol = lax.broadcasted_iota(jnp.int32, (HC, H), 1)
    sel = (hrow == hcol).astype(jnp.float32)
    alpha_self = jnp.dot(ea, sel, preferred_element_type=jnp.float32)
    ex_self = jnp.exp(alpha_self)
    denom = exs + ex_self

    brow = lax.broadcasted_iota(jnp.int32, (H, HC), 0)
    bcol = lax.broadcasted_iota(jnp.int32, (H, HC), 1) // C
    bcast = (brow == bcol).astype(jnp.float32)
    msg_tot = msg + jnp.dot(ex_self, bcast,
                            preferred_element_type=jnp.float32) * xl_ref[...]
    denb = jnp.dot(denom, bcast, preferred_element_type=jnp.float32)
    out_hc = msg_tot / (denb + 1e-16)

    crow = lax.broadcasted_iota(jnp.int32, (HC, C), 0) % C
    ccol = lax.broadcasted_iota(jnp.int32, (HC, C), 1)
    mean_m = jnp.where(crow == ccol, 1.0 / H, 0.0)
    out_ref[...] = jnp.dot(out_hc, mean_m,
                           preferred_element_type=jnp.float32) + bout_ref[...]


def _combine(acc, acc2, xl, xr, We, attf, bout2):
    grid = (N // ROW_BLK,)
    return pl.pallas_call(
        _combine_body,
        grid=grid,
        in_specs=[
            pl.BlockSpec((2, ROW_BLK, ACC_W), lambda i: (0, i, 0)),
            pl.BlockSpec((2, ROW_BLK, D_EDGE), lambda i: (0, i, 0)),
            pl.BlockSpec((ROW_BLK, HC), lambda i: (i, 0)),
            pl.BlockSpec((ROW_BLK, HC), lambda i: (i, 0)),
            pl.BlockSpec((D_EDGE, HC), lambda i: (0, 0)),
            pl.BlockSpec((1, HC), lambda i: (0, 0)),
            pl.BlockSpec((1, C), lambda i: (0, 0)),
        ],
        out_specs=pl.BlockSpec((ROW_BLK, C), lambda i: (i, 0)),
        out_shape=jax.ShapeDtypeStruct((N, C), jnp.float32),
    )(acc, acc2, xl, xr, We, attf, bout2)


def kernel(x, edge_index, relations, relation_index, Wl, bl, Wr, br, We, att,
           b_out):
    src = edge_index[0]
    dst = edge_index[1]
    bl2 = bl.reshape(1, HC)
    br2 = br.reshape(1, HC)
    attf = att.reshape(1, HC)
    bout2 = b_out.reshape(1, C)

    xl, xr, relp = _pre(x, Wl, bl2, Wr, br2, relations, We)
    acc = _edge_pass(xl, xr, relp, att, src, dst, relation_index)
    acc2 = _attr_pass(relations, dst, relation_index)
    out = _combine(acc, acc2, xl, xr, We, attf, bout2)
    return (out, relations)


# speedup vs baseline: 4.0627x; 4.0627x over previous
import functools

import jax
import jax.numpy as jnp
from jax import lax
from jax.experimental import pallas as pl
from jax.experimental.pallas import tpu as pltpu
from jax.experimental.pallas import tpu_sc as plsc

N = 10000
E = 320000
D_IN = 128
D_EDGE = 16
H = 8
C = 16
HC = H * C
N_REL = 64

ACC_W = 144
K = 80
N_CHUNKS = E // 32 // K
RPT = N // 16
ROW_BLK = 1000


def _pre_body(x_ref, wl_ref, bl_ref, wr_ref, br_ref, rel_ref, we_ref,
              xl_ref, xr_ref, relp_ref):
    xb = x_ref[...]
    xl_ref[...] = jnp.dot(xb, wl_ref[...],
                          preferred_element_type=jnp.float32) + bl_ref[...]
    xr_ref[...] = jnp.dot(xb, wr_ref[...],
                          preferred_element_type=jnp.float32) + br_ref[...]
    relp_ref[...] = jnp.dot(rel_ref[...], we_ref[...],
                            preferred_element_type=jnp.float32)


def _pre(x, Wl, bl2, Wr, br2, relations, We):
    grid = (N // ROW_BLK,)
    return pl.pallas_call(
        _pre_body,
        grid=grid,
        in_specs=[
            pl.BlockSpec((ROW_BLK, D_IN), lambda i: (i, 0)),
            pl.BlockSpec((D_IN, HC), lambda i: (0, 0)),
            pl.BlockSpec((1, HC), lambda i: (0, 0)),
            pl.BlockSpec((D_IN, HC), lambda i: (0, 0)),
            pl.BlockSpec((1, HC), lambda i: (0, 0)),
            pl.BlockSpec((N_REL, D_EDGE), lambda i: (0, 0)),
            pl.BlockSpec((D_EDGE, HC), lambda i: (0, 0)),
        ],
        out_specs=[
            pl.BlockSpec((ROW_BLK, HC), lambda i: (i, 0)),
            pl.BlockSpec((ROW_BLK, HC), lambda i: (i, 0)),
            pl.BlockSpec((N_REL, HC), lambda i: (0, 0)),
        ],
        out_shape=[
            jax.ShapeDtypeStruct((N, HC), jnp.float32),
            jax.ShapeDtypeStruct((N, HC), jnp.float32),
            jax.ShapeDtypeStruct((N_REL, HC), jnp.float32),
        ],
    )(x, Wl, bl2, Wr, br2, relations, We)


def _zero_acc(contrib, acc_sh, sid, width, rows):
    zero16 = jnp.zeros((16,), jnp.float32)

    def _zrow(r, carry):
        for j in range(width // 16):
            contrib[r, pl.ds(16 * j, 16)] = zero16
        return carry
    lax.fori_loop(0, rows, _zrow, 0)
    base = sid * RPT
    for j in range(RPT // rows):
        pltpu.sync_copy(contrib, acc_sh.at[pl.ds(base + j * rows, rows)])
    rem = RPT % rows
    if rem:
        pltpu.sync_copy(contrib.at[pl.ds(0, rem)],
                        acc_sh.at[pl.ds(base + (RPT // rows) * rows, rem)])


def _edge_body(xl_hbm, xr_hbm, relp_hbm, att_hbm,
               src_hbm, dst_hbm, rel_hbm, acc_out,
               src_v, dst_v, rel_v, xlg, xrbuf, contrib,
               relp_v, att_v, acc_sh, sema, semb, semi, semj, semk, sems):
    cid = lax.axis_index("c")
    sid = lax.axis_index("s")
    wid = cid * 16 + sid
    ebase = wid * (E // 32)
    zero16 = jnp.zeros((16,), jnp.float32)

    pltpu.sync_copy(att_hbm, att_v)
    pltpu.sync_copy(relp_hbm, relp_v)

    _zero_acc(contrib, acc_sh, sid, ACC_W, K)
    zero16i = jnp.zeros((16,), jnp.int32)
    for j in range(2):
        for b in range(K // 16):
            dst_v[j, pl.ds(16 * b, 16)] = zero16i
    plsc.subcore_barrier()

    pltpu.async_copy(src_hbm.at[pl.ds(ebase, K)], src_v.at[0], semi)
    pltpu.async_copy(dst_hbm.at[pl.ds(ebase, K)], dst_v.at[0], semj)
    pltpu.async_copy(rel_hbm.at[pl.ds(ebase, K)], rel_v.at[0], semk)
    pltpu.async_copy(contrib, acc_sh.at[dst_v.at[1]], sems, add=True)

    def _chunk(ci, carry):
        par = lax.rem(ci, 2)
        nxt = lax.rem(ci + 1, 2)
        pltpu.make_async_copy(src_hbm.at[pl.ds(ebase, K)],
                              src_v.at[par], semi).wait()
        pltpu.make_async_copy(dst_hbm.at[pl.ds(ebase, K)],
                              dst_v.at[par], semj).wait()
        pltpu.make_async_copy(rel_hbm.at[pl.ds(ebase, K)],
                              rel_v.at[par], semk).wait()
        cp_l = pltpu.async_copy(xl_hbm.at[src_v.at[par]], xlg, semb)
        cp_r = pltpu.async_copy(xr_hbm.at[dst_v.at[par]], xrbuf, sema)
        pltpu.make_async_copy(contrib, acc_sh.at[dst_v.at[nxt]], sems).wait()
        off = ebase + (ci + 1) * K
        safe = jnp.where(ci + 1 < N_CHUNKS, off, ebase)
        pltpu.async_copy(src_hbm.at[pl.ds(safe, K)], src_v.at[nxt], semi)
        pltpu.async_copy(dst_hbm.at[pl.ds(safe, K)], dst_v.at[nxt], semj)
        pltpu.async_copy(rel_hbm.at[pl.ds(safe, K)], rel_v.at[nxt], semk)
        cp_r.wait()
        cp_l.wait()

        lanes = lax.iota(jnp.int32, 16)

        def _group(g, gcarry):
            relg = rel_v[par, pl.ds(g * 16, 16)]
            for k in range(16):
                ek = g * 16 + k
                relk = relg[k]
                xs = [xlg[ek, pl.ds(16 * h, 16)] for h in range(H)]
                av = zero16
                for h in range(H):
                    ds = pl.ds(16 * h, 16)
                    e = xs[h] + xrbuf[ek, ds] + relp_v[relk, ds]
                    e = jnp.where(e < 0.0, e * 0.2, e)
                    av = jnp.where(lanes == h, jnp.sum(e * att_v[h, :]), av)
                exrow = jnp.exp(av)
                for h in range(H):
                    contrib[ek, pl.ds(16 * h, 16)] = exrow[h] * xs[h]
                contrib[ek, pl.ds(128, 16)] = exrow
            return gcarry

        lax.fori_loop(0, K // 16, _group, 0)
        pltpu.async_copy(contrib, acc_sh.at[dst_v.at[par]], sems, add=True)
        return carry

    lax.fori_loop(0, N_CHUNKS, _chunk, 0)
    pltpu.make_async_copy(contrib, acc_sh.at[dst_v.at[0]], sems).wait()
    pltpu.make_async_copy(src_hbm.at[pl.ds(ebase, K)], src_v.at[0], semi).wait()
    pltpu.make_async_copy(dst_hbm.at[pl.ds(ebase, K)], dst_v.at[0], semj).wait()
    pltpu.make_async_copy(rel_hbm.at[pl.ds(ebase, K)], rel_v.at[0], semk).wait()
    plsc.subcore_barrier()

    pltpu.sync_copy(acc_sh.at[pl.ds(sid * RPT, RPT)],
                    acc_out.at[cid, pl.ds(sid * RPT, RPT)])


def _edge_pass(xl, xr, relp, att, src, dst, rel):
    mesh = plsc.VectorSubcoreMesh(core_axis_name="c", subcore_axis_name="s")
    f = functools.partial(
        pl.kernel,
        out_type=jax.ShapeDtypeStruct((2, N, ACC_W), jnp.float32),
        mesh=mesh,
        scratch_types=[
            pltpu.VMEM((2, K), jnp.int32),
            pltpu.VMEM((2, K), jnp.int32),
            pltpu.VMEM((2, K), jnp.int32),
            pltpu.VMEM((K, HC), jnp.float32),
            pltpu.VMEM((K, HC), jnp.float32),
            pltpu.VMEM((K, ACC_W), jnp.float32),
            pltpu.VMEM((N_REL, HC), jnp.float32),
            pltpu.VMEM((H, 16), jnp.float32),
            pltpu.VMEM_SHARED((N, ACC_W), jnp.float32),
            pltpu.SemaphoreType.DMA,
            pltpu.SemaphoreType.DMA,
            pltpu.SemaphoreType.DMA,
            pltpu.SemaphoreType.DMA,
            pltpu.SemaphoreType.DMA,
            pltpu.SemaphoreType.DMA,
        ],
        compiler_params=pltpu.CompilerParams(needs_layout_passes=False,
                                             use_tc_tiling_on_sc=False),
    )(_edge_body)
    return f(xl, xr, relp, att, src, dst, rel)


K2 = 400
SCW = 80


def _attr_body(relraw_hbm, dst_hbm, rel_hbm, acc2_out,
               dst_v2, rel_v, contrib, relraw_v, acc2_sh, semi, semr, sems):
    cid = lax.axis_index("c")
    sid = lax.axis_index("s")
    wid = cid * 16 + sid
    ebase = wid * (E // 32)
    nsc = K2 // SCW
    nchunks = E // 32 // K2
    zero16 = jnp.zeros((16,), jnp.float32)
    zero16i = jnp.zeros((16,), jnp.int32)

    pltpu.sync_copy(relraw_hbm, relraw_v)
    def _zrow(r, carry):
        contrib[0, r, :] = zero16
        contrib[1, r, :] = zero16
        return carry
    lax.fori_loop(0, K2, _zrow, 0)
    for j in range(nsc):
        for b in range(SCW // 16):
            dst_v2[1, j, pl.ds(16 * b, 16)] = zero16i
    base = sid * RPT
    pltpu.sync_copy(contrib.at[0], acc2_sh.at[pl.ds(base, K2)])
    pltpu.sync_copy(contrib.at[0, pl.ds(0, RPT - K2)],
                    acc2_sh.at[pl.ds(base + K2, RPT - K2)])
    plsc.subcore_barrier()

    for j in range(nsc):
        pltpu.async_copy(dst_hbm.at[pl.ds(ebase + j * SCW, SCW)],
                         dst_v2.at[0, j], semi)
        pltpu.async_copy(contrib.at[1, pl.ds(j * SCW, SCW)],
                         acc2_sh.at[dst_v2.at[1, j]], sems, add=True)
    pltpu.async_copy(rel_hbm.at[pl.ds(ebase, K2)], rel_v.at[0], semr)

    def _chunk(ci, carry):
        par = lax.rem(ci, 2)
        nxt = lax.rem(ci + 1, 2)
        pltpu.make_async_copy(rel_hbm.at[pl.ds(ebase, K2)],
                              rel_v.at[par], semr).wait()
        for j in range(nsc):
            pltpu.make_async_copy(dst_hbm.at[pl.ds(ebase, SCW)],
                                  dst_v2.at[par, j], semi).wait()

        def _group(g, gcarry):
            relg = rel_v[par, pl.ds(g * 16, 16)]
            for k in range(16):
                contrib[par, g * 16 + k, :] = relraw_v[relg[k], :]
            return gcarry

        lax.fori_loop(0, K2 // 16, _group, 0)
        for j in range(nsc):
            pltpu.make_async_copy(contrib.at[nxt, pl.ds(j * SCW, SCW)],
                                  acc2_sh.at[dst_v2.at[nxt, j]], sems).wait()
        off = ebase + (ci + 1) * K2
        safe = jnp.where(ci + 1 < nchunks, off, ebase)
        for j in range(nsc):
            pltpu.async_copy(contrib.at[par, pl.ds(j * SCW, SCW)],
                             acc2_sh.at[dst_v2.at[par, j]], sems, add=True)
            pltpu.async_copy(dst_hbm.at[pl.ds(safe + j * SCW, SCW)],
                             dst_v2.at[nxt, j], semi)
        pltpu.async_copy(rel_hbm.at[pl.ds(safe, K2)], rel_v.at[nxt], semr)
        return carry

    lax.fori_loop(0, nchunks, _chunk, 0)
    for j in range(nsc):
        pltpu.make_async_copy(contrib.at[0, pl.ds(j * SCW, SCW)],
                              acc2_sh.at[dst_v2.at[0, j]], sems).wait()
        pltpu.make_async_copy(dst_hbm.at[pl.ds(ebase, SCW)],
                              dst_v2.at[0, j], semi).wait()
    pltpu.make_async_copy(rel_hbm.at[pl.ds(ebase, K2)],
                          rel_v.at[0], semr).wait()
    plsc.subcore_barrier()
    pltpu.sync_copy(acc2_sh.at[pl.ds(sid * RPT, RPT)],
                    acc2_out.at[cid, pl.ds(sid * RPT, RPT)])


def _attr_pass(relations, dst, rel):
    mesh = plsc.VectorSubcoreMesh(core_axis_name="c", subcore_axis_name="s")
    f = functools.partial(
        pl.kernel,
        out_type=jax.ShapeDtypeStruct((2, N, D_EDGE), jnp.float32),
        mesh=mesh,
        scratch_types=[
            pltpu.VMEM((2, K2 // SCW, SCW), jnp.int32),
            pltpu.VMEM((2, K2), jnp.int32),
            pltpu.VMEM((2, K2, D_EDGE), jnp.float32),
            pltpu.VMEM((N_REL, D_EDGE), jnp.float32),
            pltpu.VMEM_SHARED((N, D_EDGE), jnp.float32),
            pltpu.SemaphoreType.DMA,
            pltpu.SemaphoreType.DMA,
            pltpu.SemaphoreType.DMA,
        ],
        compiler_params=pltpu.CompilerParams(needs_layout_passes=False,
                                             use_tc_tiling_on_sc=False),
    )(_attr_body)
    return f(relations, dst, rel)


def _combine_body(acc_ref, acc2_ref, xl_ref, xr_ref, we_ref, attf_ref,
                  bout_ref, out_ref):
    a0 = acc_ref[0]
    a1 = acc_ref[1]
    msg = a0[:, 0:128] + a1[:, 0:128]
    exs = a0[:, 128:136] + a1[:, 128:136]
    attr = acc2_ref[0] + acc2_ref[1]
    deg = a0[:, 136:137] + a1[:, 136:137]

    loop_attr = attr / jnp.maximum(deg, 1.0)
    loop_proj = jnp.dot(loop_attr, we_ref[...],
                        preferred_element_type=jnp.float32)
    e = xl_ref[...] + xr_ref[...] + loop_proj
    e = jnp.where(e < 0.0, e * 0.2, e)
    ea = e * attf_ref[...]

    hrow = lax.broadcasted_iota(jnp.int32, (HC, H), 0) // C
    hcol = lax.broadcasted_iota(jnp.int32, (HC, H), 1)
    sel = (hrow == hcol).astype(jnp.float32)
    alpha_self = jnp.dot(ea, sel, preferred_element_type=jnp.float32)
    ex_self = jnp.exp(alpha_self)
    denom = exs + ex_self

    brow = lax.broadcasted_iota(jnp.int32, (H, HC), 0)
    bcol = lax.broadcasted_iota(jnp.int32, (H, HC), 1) // C
    bcast = (brow == bcol).astype(jnp.float32)
    msg_tot = msg + jnp.dot(ex_self, bcast,
                            preferred_element_type=jnp.float32) * xl_ref[...]
    denb = jnp.dot(denom, bcast, preferred_element_type=jnp.float32)
    out_hc = msg_tot / (denb + 1e-16)

    crow = lax.broadcasted_iota(jnp.int32, (HC, C), 0) % C
    ccol = lax.broadcasted_iota(jnp.int32, (HC, C), 1)
    mean_m = jnp.where(crow == ccol, 1.0 / H, 0.0)
    out_ref[...] = jnp.dot(out_hc, mean_m,
                           preferred_element_type=jnp.float32) + bout_ref[...]


def _combine(acc, acc2, xl, xr, We, attf, bout2):
    grid = (N // ROW_BLK,)
    return pl.pallas_call(
        _combine_body,
        grid=grid,
        in_specs=[
            pl.BlockSpec((2, ROW_BLK, ACC_W), lambda i: (0, i, 0)),
            pl.BlockSpec((2, ROW_BLK, D_EDGE), lambda i: (0, i, 0)),
            pl.BlockSpec((ROW_BLK, HC), lambda i: (i, 0)),
            pl.BlockSpec((ROW_BLK, HC), lambda i: (i, 0)),
            pl.BlockSpec((D_EDGE, HC), lambda i: (0, 0)),
            pl.BlockSpec((1, HC), lambda i: (0, 0)),
            pl.BlockSpec((1, C), lambda i: (0, 0)),
        ],
        out_specs=pl.BlockSpec((ROW_BLK, C), lambda i: (i, 0)),
        out_shape=jax.ShapeDtypeStruct((N, C), jnp.float32),
    )(acc, acc2, xl, xr, We, attf, bout2)


def kernel(x, edge_index, relations, relation_index, Wl, bl, Wr, br, We, att,
           b_out):
    src = edge_index[0]
    dst = edge_index[1]
    bl2 = bl.reshape(1, HC)
    br2 = br.reshape(1, HC)
    attf = att.reshape(1, HC)
    bout2 = b_out.reshape(1, C)

    xl, xr, relp = _pre(x, Wl, bl2, Wr, br2, relations, We)
    acc = _edge_pass(xl, xr, relp, att, src, dst, relation_index)
    acc2 = _attr_pass(relations, dst, relation_index)
    out = _combine(acc, acc2, xl, xr, We, attf, bout2)
    return (out, relations)
